# baseline (speedup 1.0000x reference)
import functools
import math
import random

import numpy as np
import jax
import jax.numpy as jnp
from jax import lax
from jax.experimental import pallas as pl
from jax.experimental.pallas import tpu as pltpu
from jax.experimental.pallas import tpu_sc as plsc

LANES = 16
NBUF = 4
TC_BLOCK = 16


def _distortion_indices(h, w, a_max=3, r_max=0.7):
    random.seed(0)
    cols = h
    rows = w
    center_rows = int(np.round(random.uniform(1, rows - 2)))
    center_cols = int(np.round(random.uniform(1, cols - 2)))
    radius = random.uniform(0.03 * max(rows, cols), r_max * max(rows, cols))
    choice = random.randint(0, 1)
    spect_ratio1 = 1
    spect_ratio2 = 1
    if choice == 1:
        spect_ratio1 = random.uniform(1, a_max)
    else:
        spect_ratio2 = random.uniform(1, a_max)
    cols_np = np.arange(cols)
    rows_np = np.arange(rows)
    cols_np_t = np.tile(cols_np, (rows, 1))
    cols_pow = np.power(cols_np_t - center_cols, 2)
    rows_np_t = np.tile(rows_np, (cols, 1))
    rows_pow = np.power(rows_np_t - center_rows, 2)
    dis = np.sqrt(cols_pow + rows_pow.transpose())
    judge = (spect_ratio1 * np.abs(rows_np_t - center_rows).transpose()
             + spect_ratio2 * np.abs(cols_np_t - center_cols))
    index = np.where(judge <= radius)
    index_rows = np.rint(index[0]).astype(np.int64)
    index_cols = np.rint(index[1]).astype(np.int64)
    dis_val = dis[index]
    old_i = np.floor(dis_val / radius * (index_rows - center_rows) + center_rows)
    old_j = np.floor(dis_val / radius * (index_cols - center_cols) + center_cols)
    return (index_rows, index_cols,
            old_i.astype(np.int64), old_j.astype(np.int64))


def _band_patch_indices(h, w):
    ir, ic, oi, oj = _distortion_indices(h, w)
    oi = np.where(oi < 0, oi + h, oi).clip(0, h - 1)
    oj = np.where(oj < 0, oj + w, oj).clip(0, w - 1)
    r0 = int(min(ir.min(), oi.min())) // 8 * 8
    r1 = -(-(int(max(ir.max(), oi.max())) + 1) // 8) * 8
    nr = r1 - r0
    src = ((oi - r0) * w + oj).astype(np.int32)
    dst = ((ir - r0) * w + ic).astype(np.int32)
    k = src.shape[0]
    k_pad = math.ceil(k / LANES) * LANES
    src = np.concatenate([src, np.full(k_pad - k, src[-1], np.int32)])
    dst = np.concatenate([dst, np.full(k_pad - k, dst[-1], np.int32)])
    return r0, nr, src, dst


@functools.cache
def _build_sc_band_call(n_slices, h, w, r0, nr, k_pad):
    info = plsc.get_sparse_core_info()
    nc, ns = info.num_cores, info.num_subcores
    n_workers = nc * ns
    assert n_slices % n_workers == 0
    per_worker = n_slices // n_workers
    assert per_worker % NBUF == 0
    n_chunks = k_pad // LANES
    bw = nr * w
    mesh = plsc.VectorSubcoreMesh(core_axis_name="c", subcore_axis_name="s")

    @functools.partial(
        pl.kernel,
        mesh=mesh,
        out_type=jax.ShapeDtypeStruct((n_slices, bw), jnp.float32),
        compiler_params=pltpu.CompilerParams(needs_layout_passes=False),
        scratch_types=[
            pltpu.VMEM((k_pad,), jnp.int32),
            pltpu.VMEM((k_pad,), jnp.int32),
            pltpu.VMEM((k_pad,), jnp.float32),
        ] + [pltpu.VMEM((bw,), jnp.float32) for _ in range(NBUF)]
          + [pltpu.SemaphoreType.DMA for _ in range(2 * NBUF)],
    )
    def sc_band(feat_hbm, src_hbm, dst_hbm, band_hbm,
                src_v, dst_v, vals_v, *bufs_and_sems):
        bufs = bufs_and_sems[:NBUF]
        lsems = bufs_and_sems[NBUF:2 * NBUF]
        ssems = bufs_and_sems[2 * NBUF:]
        wid = lax.axis_index("s") * nc + lax.axis_index("c")
        sl0 = wid * per_worker
        pltpu.sync_copy(src_hbm, src_v)
        pltpu.sync_copy(dst_hbm, dst_v)

        def start_load(sl, b):
            off = pl.multiple_of(sl * (h * w) + r0 * w, 8)
            pltpu.async_copy(feat_hbm.at[pl.ds(off, bw)], bufs[b],
                             lsems[b])

        def wait_load(b):
            pltpu.make_async_copy(feat_hbm.at[pl.ds(0, bw)], bufs[b],
                                  lsems[b]).wait()

        def start_store(sl, b):
            pltpu.async_copy(bufs[b], band_hbm.at[sl], ssems[b])

        def wait_store(b):
            pltpu.make_async_copy(bufs[b], band_hbm.at[sl0],
                                  ssems[b]).wait()

        for b in range(NBUF):
            start_load(sl0 + b, b)

        def do_group(g, _):
            for b in range(NBUF):
                i = g * NBUF + b
                buf = bufs[b]
                wait_load(b)

                def gather_chunk(t, _):
                    sel = pl.ds(t * LANES, LANES)
                    vals_v[sel] = plsc.load_gather(buf, [src_v[sel]])
                    return 0

                lax.fori_loop(0, n_chunks, gather_chunk, 0, unroll=8)

                def scatter_chunk(t, _):
                    sel = pl.ds(t * LANES, LANES)
                    plsc.store_scatter(buf, [dst_v[sel]], vals_v[sel])
                    return 0

                lax.fori_loop(0, n_chunks, scatter_chunk, 0, unroll=8)
                start_store(sl0 + i, b)

                @pl.when(i + NBUF < per_worker)
                def _prefetch():
                    wait_store(b)
                    start_load(sl0 + i + NBUF, b)
            return 0

        lax.fori_loop(0, per_worker // NBUF, do_group, 0)
        for b in range(NBUF):
            wait_store(b)

    return sc_band


@functools.cache
def _build_tc_paste_call(n_slices, h, w, r0, nr):
    n_hi = h - r0 - nr

    def tc_body(feat_ref, band_ref, out_ref):
        out_ref[:, :r0, :] = feat_ref[:, :r0, :]
        out_ref[:, r0:r0 + nr, :] = band_ref[...]
        out_ref[:, r0 + nr:, :] = feat_ref[:, r0 + nr:, :]

    grid = (n_slices // TC_BLOCK,)
    return pl.pallas_call(
        tc_body,
        grid=grid,
        in_specs=[
            pl.BlockSpec((TC_BLOCK, h, w), lambda i: (i, 0, 0)),
            pl.BlockSpec((TC_BLOCK, nr, w), lambda i: (i, 0, 0)),
        ],
        out_specs=pl.BlockSpec((TC_BLOCK, h, w), lambda i: (i, 0, 0)),
        out_shape=jax.ShapeDtypeStruct((n_slices, h, w), jnp.float32),
        compiler_params=pltpu.CompilerParams(
            dimension_semantics=("arbitrary",)),
    )


def kernel(feature):
    b, c, h, w = feature.shape
    r0, nr, src, dst = _band_patch_indices(h, w)
    n_slices = b * c
    feat3 = feature.reshape(n_slices, h, w)
    sc_band = _build_sc_band_call(n_slices, h, w, r0, nr, src.shape[0])
    band = sc_band(feature.reshape(n_slices * h * w),
                   jnp.asarray(src), jnp.asarray(dst))
    tc_paste = _build_tc_paste_call(n_slices, h, w, r0, nr)
    out = tc_paste(feat3, band.reshape(n_slices, nr, w))
    return out.reshape(b, c, h, w)


# speedup vs baseline: 1.4682x; 1.4682x over previous
import functools
import math
import random

import numpy as np
import jax
import jax.numpy as jnp
from jax import lax
from jax.experimental import pallas as pl
from jax.experimental.pallas import tpu as pltpu
from jax.experimental.pallas import tpu_sc as plsc

LANES = 16
NBUF = 4
TC_BLOCK = 16


def _distortion_indices(h, w, a_max=3, r_max=0.7):
    random.seed(0)
    cols = h
    rows = w
    center_rows = int(np.round(random.uniform(1, rows - 2)))
    center_cols = int(np.round(random.uniform(1, cols - 2)))
    radius = random.uniform(0.03 * max(rows, cols), r_max * max(rows, cols))
    choice = random.randint(0, 1)
    spect_ratio1 = 1
    spect_ratio2 = 1
    if choice == 1:
        spect_ratio1 = random.uniform(1, a_max)
    else:
        spect_ratio2 = random.uniform(1, a_max)
    cols_np = np.arange(cols)
    rows_np = np.arange(rows)
    cols_np_t = np.tile(cols_np, (rows, 1))
    cols_pow = np.power(cols_np_t - center_cols, 2)
    rows_np_t = np.tile(rows_np, (cols, 1))
    rows_pow = np.power(rows_np_t - center_rows, 2)
    dis = np.sqrt(cols_pow + rows_pow.transpose())
    judge = (spect_ratio1 * np.abs(rows_np_t - center_rows).transpose()
             + spect_ratio2 * np.abs(cols_np_t - center_cols))
    index = np.where(judge <= radius)
    index_rows = np.rint(index[0]).astype(np.int64)
    index_cols = np.rint(index[1]).astype(np.int64)
    dis_val = dis[index]
    old_i = np.floor(dis_val / radius * (index_rows - center_rows) + center_rows)
    old_j = np.floor(dis_val / radius * (index_cols - center_cols) + center_cols)
    return (index_rows, index_cols,
            old_i.astype(np.int64), old_j.astype(np.int64))


def _band_patch_indices(h, w):
    ir, ic, oi, oj = _distortion_indices(h, w)
    oi = np.where(oi < 0, oi + h, oi).clip(0, h - 1)
    oj = np.where(oj < 0, oj + w, oj).clip(0, w - 1)
    r0 = int(min(ir.min(), oi.min())) // 8 * 8
    r1 = -(-(int(max(ir.max(), oi.max())) + 1) // 8) * 8
    nr = r1 - r0
    k = ir.shape[0]
    k_pad = math.ceil(k / LANES) * LANES
    pad = lambda a: np.concatenate(
        [a, np.full(k_pad - k, a[-1])]).astype(np.int32)
    return r0, nr, pad(oi - r0), pad(oj), pad(ir - r0), pad(ic)


@functools.cache
def _build_sc_band_call(n_slices, h, w, r0, nr, k_pad):
    info = plsc.get_sparse_core_info()
    nc, ns = info.num_cores, info.num_subcores
    n_workers = nc * ns
    assert n_slices % n_workers == 0
    per_worker = n_slices // n_workers
    assert per_worker % NBUF == 0
    n_chunks = k_pad // LANES
    mesh = plsc.VectorSubcoreMesh(core_axis_name="c", subcore_axis_name="s")

    @functools.partial(
        pl.kernel,
        mesh=mesh,
        out_type=jax.ShapeDtypeStruct((n_slices, nr, w), jnp.float32),
        compiler_params=pltpu.CompilerParams(needs_layout_passes=False),
        scratch_types=[
            pltpu.VMEM((k_pad,), jnp.int32),
            pltpu.VMEM((k_pad,), jnp.int32),
            pltpu.VMEM((k_pad,), jnp.int32),
            pltpu.VMEM((k_pad,), jnp.int32),
            pltpu.VMEM((k_pad,), jnp.float32),
        ] + [pltpu.VMEM((nr, w), jnp.float32) for _ in range(NBUF)]
          + [pltpu.SemaphoreType.DMA for _ in range(2 * NBUF)],
    )
    def sc_band(feat_hbm, gr_hbm, gc_hbm, dr_hbm, dc_hbm, band_hbm,
                gr_v, gc_v, dr_v, dc_v, vals_v, *bufs_and_sems):
        bufs = bufs_and_sems[:NBUF]
        lsems = bufs_and_sems[NBUF:2 * NBUF]
        ssems = bufs_and_sems[2 * NBUF:]
        wid = lax.axis_index("s") * nc + lax.axis_index("c")
        sl0 = wid * per_worker
        pltpu.sync_copy(gr_hbm, gr_v)
        pltpu.sync_copy(gc_hbm, gc_v)
        pltpu.sync_copy(dr_hbm, dr_v)
        pltpu.sync_copy(dc_hbm, dc_v)

        def start_load(sl, b):
            pltpu.async_copy(feat_hbm.at[sl, pl.ds(r0, nr)], bufs[b],
                             lsems[b])

        def wait_load(b):
            pltpu.make_async_copy(feat_hbm.at[sl0, pl.ds(r0, nr)], bufs[b],
                                  lsems[b]).wait()

        def start_store(sl, b):
            pltpu.async_copy(bufs[b], band_hbm.at[sl], ssems[b])

        def wait_store(b):
            pltpu.make_async_copy(bufs[b], band_hbm.at[sl0],
                                  ssems[b]).wait()

        for b in range(NBUF):
            start_load(sl0 + b, b)

        def do_group(g, _):
            for b in range(NBUF):
                i = g * NBUF + b
                buf = bufs[b]
                wait_load(b)

                def gather_chunk(t, _):
                    sel = pl.ds(t * LANES, LANES)
                    vals_v[sel] = plsc.load_gather(
                        buf, [gr_v[sel], gc_v[sel]])
                    return 0

                lax.fori_loop(0, n_chunks, gather_chunk, 0, unroll=8)

                def scatter_chunk(t, _):
                    sel = pl.ds(t * LANES, LANES)
                    plsc.store_scatter(buf, [dr_v[sel], dc_v[sel]],
                                       vals_v[sel])
                    return 0

                lax.fori_loop(0, n_chunks, scatter_chunk, 0, unroll=8)
                start_store(sl0 + i, b)

                @pl.when(i + NBUF < per_worker)
                def _prefetch():
                    wait_store(b)
                    start_load(sl0 + i + NBUF, b)
            return 0

        lax.fori_loop(0, per_worker // NBUF, do_group, 0)
        for b in range(NBUF):
            wait_store(b)

    return sc_band


@functools.cache
def _build_tc_paste_call(n_slices, h, w, r0, nr):
    n_hi = h - r0 - nr

    def tc_body(feat_ref, band_ref, out_ref):
        out_ref[:, :r0, :] = feat_ref[:, :r0, :]
        out_ref[:, r0:r0 + nr, :] = band_ref[...]
        out_ref[:, r0 + nr:, :] = feat_ref[:, r0 + nr:, :]

    grid = (n_slices // TC_BLOCK,)
    return pl.pallas_call(
        tc_body,
        grid=grid,
        in_specs=[
            pl.BlockSpec((TC_BLOCK, h, w), lambda i: (i, 0, 0)),
            pl.BlockSpec((TC_BLOCK, nr, w), lambda i: (i, 0, 0)),
        ],
        out_specs=pl.BlockSpec((TC_BLOCK, h, w), lambda i: (i, 0, 0)),
        out_shape=jax.ShapeDtypeStruct((n_slices, h, w), jnp.float32),
        compiler_params=pltpu.CompilerParams(
            dimension_semantics=("arbitrary",)),
    )


def kernel(feature):
    b, c, h, w = feature.shape
    r0, nr, gr, gc, dr, dc = _band_patch_indices(h, w)
    n_slices = b * c
    feat3 = feature.reshape(n_slices, h, w)
    sc_band = _build_sc_band_call(n_slices, h, w, r0, nr, gr.shape[0])
    band = sc_band(feat3, jnp.asarray(gr), jnp.asarray(gc),
                   jnp.asarray(dr), jnp.asarray(dc))
    tc_paste = _build_tc_paste_call(n_slices, h, w, r0, nr)
    out = tc_paste(feat3, band)
    return out.reshape(b, c, h, w)
